# baseline (speedup 1.0000x reference)
import jax
import jax.numpy as jnp
from jax import lax
from jax.experimental import pallas as pl
from jax.experimental.pallas import tpu as pltpu
from jax.experimental.pallas import tpu_sc as plsc

K = 1024
HW = 128 * 256
NBKT = 4096
BKT_SHIFT = 20
ONE_F32_BITS = 0x3F800000


def _key_body(cav_ref, ego_ref, out_ref):
    cav = cav_ref[0, 0]
    ego = ego_ref[0, 0]
    r = jnp.maximum(cav[0] - ego[0], cav[1] - ego[1])
    bits = lax.bitcast_convert_type(r, jnp.uint32)
    neg = bits >= jnp.uint32(0x80000000)
    out_ref[0] = jnp.where(neg, ~bits, bits | jnp.uint32(0x80000000))


def _scan_hist(hist_ref, nbkt, need):
    lane = lax.iota(jnp.int32, 16)
    nvr = nbkt // 16
    ngrp = nvr // 16
    z = jnp.int32(0)

    def gstep(gg, carry):
        T, gstar, Tstar, found = carry
        g = (ngrp - 1) - gg
        acc = jnp.zeros((16,), jnp.int32)
        for t in range(16):
            acc = acc + hist_ref[pl.ds((g * 16 + t) * 16, 16)]
        s = jnp.sum(acc)
        cross = (1 - found) * jnp.where(T + s >= need, 1, 0)
        return (T + s,
                jnp.where(cross == 1, g, gstar),
                jnp.where(cross == 1, T, Tstar),
                jnp.where(T + s >= need, 1, found))

    _, gstar, Tg, _ = lax.fori_loop(0, ngrp, gstep, (z, z, z, z))

    def vstep(tt, carry):
        T, jstar, Tstar, found = carry
        j = gstar * 16 + (15 - tt)
        v = hist_ref[pl.ds(j * 16, 16)]
        s = jnp.sum(v)
        cross = (1 - found) * jnp.where(T + s >= need, 1, 0)
        return (T + s,
                jnp.where(cross == 1, j, jstar),
                jnp.where(cross == 1, T, Tstar),
                jnp.where(T + s >= need, 1, found))

    _, jstar, Tstar, _ = lax.fori_loop(0, 16, vstep, (Tg, z, z, z))

    v = hist_ref[pl.ds(jstar * 16, 16)]
    cs = plsc.cumsum(lax.rev(v, (0,)))
    m0 = jnp.min(jnp.where(cs >= need - Tstar, lane, 16))
    i = 15 - m0
    cge = Tstar + jnp.max(jnp.where(lane == m0, cs, 0))
    M = jnp.max(jnp.where(lane == i, v, 0))
    Bx = jstar * 16 + i
    return Bx, cge - M


def _sc_mask_body(keys_hbm, out_hbm, key_v, h1_v, cbuf_v, cnt_v):
    w = lax.axis_index("s") * 2 + lax.axis_index("c")

    @pl.when(w < 16)
    def _():
        zero16_i = jnp.zeros((16,), jnp.int32)
        ones16_i = jnp.ones((16,), jnp.int32)
        NV = HW // 16
        U = 8

        def zh(i, carry):
            h1_v[pl.ds(i * 16, 16)] = zero16_i
            return carry

        lax.fori_loop(0, NBKT // 16, zh, 0)

        pltpu.sync_copy(keys_hbm.at[w], key_v)

        def hpass1(jo, carry):
            for u in range(U):
                key = key_v[pl.ds((jo * U + u) * 16, 16)]
                bkt = (key >> jnp.uint32(BKT_SHIFT)).astype(jnp.int32)
                plsc.addupdate_scatter(h1_v, [bkt], ones16_i)
            return carry

        lax.fori_loop(0, NV // U, hpass1, 0)
        B1, gt1 = _scan_hist(h1_v, NBKT, K)

        lane = lax.iota(jnp.int32, 16)

        def apass(jg, carry):
            acc = jnp.zeros((16,), jnp.int32)
            for u in range(16):
                key = key_v[pl.ds((jg * 16 + u) * 16, 16)]
                m = (key >> jnp.uint32(BKT_SHIFT)).astype(jnp.int32) == B1
                pc = plsc.all_reduce_population_count(m)
                acc = jnp.where(lane == u, pc, acc)
            cnt_v[pl.ds(jg * 16, 16)] = acc
            return carry

        lax.fori_loop(0, NV // 16, apass, 0)

        def bpass(jg, off):
            cv = cnt_v[pl.ds(jg * 16, 16)]
            gcnt = jnp.sum(cv)

            def visit(off2):
                def one(u, off3):
                    j = jg * 16 + u
                    key = key_v[pl.ds(j * 16, 16)]
                    m = ((key >> jnp.uint32(BKT_SHIFT)).astype(jnp.int32)
                         == B1)
                    cnt = plsc.all_reduce_population_count(m)[0]
                    plsc.store_compressed(cbuf_v.at[pl.ds(off3, 16)], key,
                                          mask=m)
                    return off3 + cnt

                return lax.fori_loop(0, 16, one, off2)

            return lax.cond(gcnt > 0, visit, lambda o: o, off)

        M = lax.fori_loop(0, NV // 16 // 16, lambda gg, off: lax.cond(
            jnp.sum(lax.fori_loop(
                0, 16, lambda t, a: a + cnt_v[pl.ds((gg * 16 + t) * 16, 16)],
                jnp.zeros((16,), jnp.int32))) > 0,
            lambda o: lax.fori_loop(gg * 16, gg * 16 + 16, bpass, o),
            lambda o: o, off), jnp.int32(0))
        cbuf_v[pl.ds(M, 16)] = jnp.zeros((16,), jnp.uint32)

        nv = (M + 15) // 16
        prefix0 = B1.astype(jnp.uint32) << jnp.uint32(BKT_SHIFT)

        def refine(it, prefix):
            cand = prefix | (jnp.uint32(1) << (jnp.uint32(BKT_SHIFT - 1) -
                                               it.astype(jnp.uint32)))

            def cntloop(j, acc):
                kv = cbuf_v[pl.ds(j * 16, 16)]
                return acc + jnp.where(kv >= cand, 1, 0)

            accv = lax.fori_loop(0, nv, cntloop, jnp.zeros((16,), jnp.int32))
            cnt = jnp.sum(accv) + gt1
            return jnp.where(cnt >= K, cand, prefix)

        thr = lax.fori_loop(0, BKT_SHIFT, refine, prefix0)

        def mpass(jo, carry):
            for u in range(U):
                s = pl.ds((jo * U + u) * 16, 16)
                key = key_v[s]
                key_v[s] = jnp.where(key >= thr, jnp.uint32(ONE_F32_BITS),
                                     jnp.uint32(0))
            return carry

        lax.fori_loop(0, NV // U, mpass, 0)
        pltpu.sync_copy(key_v, out_hbm.at[w])


def _sc_masks(keys):
    f = pl.kernel(
        _sc_mask_body,
        out_type=jax.ShapeDtypeStruct((16, HW), jnp.uint32),
        mesh=plsc.VectorSubcoreMesh(core_axis_name="c", subcore_axis_name="s",
                                    num_cores=2, num_subcores=16),
        compiler_params=pltpu.CompilerParams(needs_layout_passes=False),
        scratch_types=[
            pltpu.VMEM((HW,), jnp.uint32),
            pltpu.VMEM((NBKT,), jnp.int32),
            pltpu.VMEM((HW + 16,), jnp.uint32),
            pltpu.VMEM((HW // 16,), jnp.int32),
        ],
    )
    return f(keys)


def _apply_body(mask_ref, x_ref, m_ref, o_ref):
    p = pl.program_id(0)
    b = p // 5
    l = p - b * 5
    xv = x_ref[...]
    mplane = m_ref[...]
    mv = (mask_ref[b, l] != 0).astype(jnp.float32)
    o_ref[...] = jnp.where(l == 0, xv, xv * (mplane * mv)[:, None, :, :])


def kernel(x, psm, mask):
    B, L, C, H, W = x.shape

    keys = pl.pallas_call(
        _key_body,
        grid=(B * (L - 1),),
        in_specs=[
            pl.BlockSpec((1, 1, 2, H, W), lambda p: (p // 4, p % 4 + 1, 0, 0, 0)),
            pl.BlockSpec((1, 1, 2, H, W), lambda p: (p // 4, 0, 0, 0, 0)),
        ],
        out_specs=pl.BlockSpec((1, H, W), lambda p: (p, 0, 0)),
        out_shape=jax.ShapeDtypeStruct((B * (L - 1), H, W), jnp.uint32),
    )(psm, psm)

    mask16 = _sc_masks(keys.reshape(B * (L - 1), H * W))
    mplane = lax.bitcast_convert_type(mask16, jnp.float32).reshape(16, H, W)

    CB = 16
    xf = x.reshape(B * L, C, H, W)
    out = pl.pallas_call(
        _apply_body,
        grid=(B * L, C // CB),
        in_specs=[
            pl.BlockSpec(memory_space=pltpu.SMEM),
            pl.BlockSpec((1, CB, H, W), lambda p, c: (p, c, 0, 0)),
            pl.BlockSpec((1, H, W),
                         lambda p, c: (jnp.clip(p - p // 5 - 1, 0, 15), 0, 0)),
        ],
        out_specs=pl.BlockSpec((1, CB, H, W), lambda p, c: (p, c, 0, 0)),
        out_shape=jax.ShapeDtypeStruct((B * L, C, H, W), jnp.float32),
    )(mask, xf, mplane)
    return out.reshape(B, L, C, H, W)


# speedup vs baseline: 1.1006x; 1.1006x over previous
import jax
import jax.numpy as jnp
from jax import lax
from jax.experimental import pallas as pl
from jax.experimental.pallas import tpu as pltpu
from jax.experimental.pallas import tpu_sc as plsc

K = 1024
HW = 128 * 256
NBKT = 4096
BKT_SHIFT = 20
ONE_F32_BITS = 0x3F800000


def _key_body(cav_ref, ego_ref, out_ref):
    cav = cav_ref[0, 0]
    ego = ego_ref[0, 0]
    r = jnp.maximum(cav[0] - ego[0], cav[1] - ego[1])
    bits = lax.bitcast_convert_type(r, jnp.uint32)
    neg = bits >= jnp.uint32(0x80000000)
    out_ref[0] = jnp.where(neg, ~bits, bits | jnp.uint32(0x80000000))


def _scan_hist(hist_ref, nbkt, need):
    lane = lax.iota(jnp.int32, 16)
    nvr = nbkt // 16
    ngrp = nvr // 16
    z = jnp.int32(0)

    def gstep(gg, carry):
        T, gstar, Tstar, found = carry
        g = (ngrp - 1) - gg
        acc = jnp.zeros((16,), jnp.int32)
        for t in range(16):
            acc = acc + hist_ref[pl.ds((g * 16 + t) * 16, 16)]
        s = jnp.sum(acc)
        cross = (1 - found) * jnp.where(T + s >= need, 1, 0)
        return (T + s,
                jnp.where(cross == 1, g, gstar),
                jnp.where(cross == 1, T, Tstar),
                jnp.where(T + s >= need, 1, found))

    _, gstar, Tg, _ = lax.fori_loop(0, ngrp, gstep, (z, z, z, z))

    def vstep(tt, carry):
        T, jstar, Tstar, found = carry
        j = gstar * 16 + (15 - tt)
        v = hist_ref[pl.ds(j * 16, 16)]
        s = jnp.sum(v)
        cross = (1 - found) * jnp.where(T + s >= need, 1, 0)
        return (T + s,
                jnp.where(cross == 1, j, jstar),
                jnp.where(cross == 1, T, Tstar),
                jnp.where(T + s >= need, 1, found))

    _, jstar, Tstar, _ = lax.fori_loop(0, 16, vstep, (Tg, z, z, z))

    v = hist_ref[pl.ds(jstar * 16, 16)]
    cs = plsc.cumsum(lax.rev(v, (0,)))
    m0 = jnp.min(jnp.where(cs >= need - Tstar, lane, 16))
    i = 15 - m0
    cge = Tstar + jnp.max(jnp.where(lane == m0, cs, 0))
    M = jnp.max(jnp.where(lane == i, v, 0))
    Bx = jstar * 16 + i
    return Bx, cge - M


def _sc_mask_body(keys_hbm, out_hbm, key_v, h1_v, cbuf_v):
    w = lax.axis_index("s") * 2 + lax.axis_index("c")

    @pl.when(w < 16)
    def _():
        zero16_i = jnp.zeros((16,), jnp.int32)
        ones16_i = jnp.ones((16,), jnp.int32)
        NV = HW // 16
        U = 8

        def zh(i, carry):
            h1_v[pl.ds(i * 16, 16)] = zero16_i
            return carry

        lax.fori_loop(0, NBKT // 16, zh, 0)

        pltpu.sync_copy(keys_hbm.at[w], key_v)

        def hpass1(jo, carry):
            for u in range(U):
                key = key_v[pl.ds((jo * U + u) * 16, 16)]
                bkt = (key >> jnp.uint32(BKT_SHIFT)).astype(jnp.int32)
                plsc.addupdate_scatter(h1_v, [bkt], ones16_i)
            return carry

        lax.fori_loop(0, NV // U, hpass1, 0)
        B1, gt1 = _scan_hist(h1_v, NBKT, K)

        def cpass(jo, off):
            for u in range(U):
                key = key_v[pl.ds((jo * U + u) * 16, 16)]
                m = (key >> jnp.uint32(BKT_SHIFT)).astype(jnp.int32) == B1
                cnt = plsc.all_reduce_population_count(m)[0]
                plsc.store_compressed(cbuf_v.at[pl.ds(off, 16)], key, mask=m)
                off = off + cnt
            return off

        M = lax.fori_loop(0, NV // U, cpass, jnp.int32(0))
        cbuf_v[pl.ds(M, 16)] = jnp.zeros((16,), jnp.uint32)

        nv = (M + 15) // 16
        prefix0 = B1.astype(jnp.uint32) << jnp.uint32(BKT_SHIFT)

        def refine(it, prefix):
            cand = prefix | (jnp.uint32(1) << (jnp.uint32(BKT_SHIFT - 1) -
                                               it.astype(jnp.uint32)))

            def cntloop(j, acc):
                kv = cbuf_v[pl.ds(j * 16, 16)]
                return acc + jnp.where(kv >= cand, 1, 0)

            accv = lax.fori_loop(0, nv, cntloop, jnp.zeros((16,), jnp.int32))
            cnt = jnp.sum(accv) + gt1
            return jnp.where(cnt >= K, cand, prefix)

        thr = lax.fori_loop(0, BKT_SHIFT, refine, prefix0)

        def mpass(jo, carry):
            for u in range(U):
                s = pl.ds((jo * U + u) * 16, 16)
                key = key_v[s]
                key_v[s] = jnp.where(key >= thr, jnp.uint32(ONE_F32_BITS),
                                     jnp.uint32(0))
            return carry

        lax.fori_loop(0, NV // U, mpass, 0)
        pltpu.sync_copy(key_v, out_hbm.at[w])


def _sc_masks(keys):
    f = pl.kernel(
        _sc_mask_body,
        out_type=jax.ShapeDtypeStruct((16, HW), jnp.uint32),
        mesh=plsc.VectorSubcoreMesh(core_axis_name="c", subcore_axis_name="s",
                                    num_cores=2, num_subcores=16),
        compiler_params=pltpu.CompilerParams(needs_layout_passes=False),
        scratch_types=[
            pltpu.VMEM((HW,), jnp.uint32),
            pltpu.VMEM((NBKT,), jnp.int32),
            pltpu.VMEM((HW + 16,), jnp.uint32),
        ],
    )
    return f(keys)


def _apply_body(mask_ref, x_ref, m_ref, o_ref):
    p = pl.program_id(0)
    b = p // 5
    l = p - b * 5
    xv = x_ref[...]
    mplane = m_ref[...]
    mv = (mask_ref[b, l] != 0).astype(jnp.float32)
    o_ref[...] = jnp.where(l == 0, xv, xv * (mplane * mv)[:, None, :, :])


def kernel(x, psm, mask):
    B, L, C, H, W = x.shape

    keys = pl.pallas_call(
        _key_body,
        grid=(B * (L - 1),),
        in_specs=[
            pl.BlockSpec((1, 1, 2, H, W), lambda p: (p // 4, p % 4 + 1, 0, 0, 0)),
            pl.BlockSpec((1, 1, 2, H, W), lambda p: (p // 4, 0, 0, 0, 0)),
        ],
        out_specs=pl.BlockSpec((1, H, W), lambda p: (p, 0, 0)),
        out_shape=jax.ShapeDtypeStruct((B * (L - 1), H, W), jnp.uint32),
    )(psm, psm)

    mask16 = _sc_masks(keys.reshape(B * (L - 1), H * W))
    mplane = lax.bitcast_convert_type(mask16, jnp.float32).reshape(16, H, W)

    CB = 32
    xf = x.reshape(B * L, C, H, W)
    out = pl.pallas_call(
        _apply_body,
        grid=(B * L, C // CB),
        in_specs=[
            pl.BlockSpec(memory_space=pltpu.SMEM),
            pl.BlockSpec((1, CB, H, W), lambda p, c: (p, c, 0, 0)),
            pl.BlockSpec((1, H, W),
                         lambda p, c: (jnp.clip(p - p // 5 - 1, 0, 15), 0, 0)),
        ],
        out_specs=pl.BlockSpec((1, CB, H, W), lambda p, c: (p, c, 0, 0)),
        out_shape=jax.ShapeDtypeStruct((B * L, C, H, W), jnp.float32),
    )(mask, xf, mplane)
    return out.reshape(B, L, C, H, W)


# speedup vs baseline: 1.1085x; 1.0072x over previous
import jax
import jax.numpy as jnp
from jax import lax
from jax.experimental import pallas as pl
from jax.experimental.pallas import tpu as pltpu
from jax.experimental.pallas import tpu_sc as plsc

K = 1024
HW = 128 * 256
NBKT = 4096
BKT_SHIFT = 20
ONE_F32_BITS = 0x3F800000


def _key_body(cav_ref, ego_ref, out_ref):
    cav = cav_ref[0, 0]
    ego = ego_ref[0, 0]
    r = jnp.maximum(cav[0] - ego[0], cav[1] - ego[1])
    bits = lax.bitcast_convert_type(r, jnp.uint32)
    neg = bits >= jnp.uint32(0x80000000)
    out_ref[0] = jnp.where(neg, ~bits, bits | jnp.uint32(0x80000000))


def _scan_hist(hist_ref, nbkt, need):
    lane = lax.iota(jnp.int32, 16)
    nvr = nbkt // 16
    ngrp = nvr // 16
    z = jnp.int32(0)

    def gstep(gg, carry):
        T, gstar, Tstar, found = carry
        g = (ngrp - 1) - gg
        acc = jnp.zeros((16,), jnp.int32)
        for t in range(16):
            acc = acc + hist_ref[pl.ds((g * 16 + t) * 16, 16)]
        s = jnp.sum(acc)
        cross = (1 - found) * jnp.where(T + s >= need, 1, 0)
        return (T + s,
                jnp.where(cross == 1, g, gstar),
                jnp.where(cross == 1, T, Tstar),
                jnp.where(T + s >= need, 1, found))

    _, gstar, Tg, _ = lax.fori_loop(0, ngrp, gstep, (z, z, z, z))

    def vstep(tt, carry):
        T, jstar, Tstar, found = carry
        j = gstar * 16 + (15 - tt)
        v = hist_ref[pl.ds(j * 16, 16)]
        s = jnp.sum(v)
        cross = (1 - found) * jnp.where(T + s >= need, 1, 0)
        return (T + s,
                jnp.where(cross == 1, j, jstar),
                jnp.where(cross == 1, T, Tstar),
                jnp.where(T + s >= need, 1, found))

    _, jstar, Tstar, _ = lax.fori_loop(0, 16, vstep, (Tg, z, z, z))

    v = hist_ref[pl.ds(jstar * 16, 16)]
    cs = plsc.cumsum(lax.rev(v, (0,)))
    m0 = jnp.min(jnp.where(cs >= need - Tstar, lane, 16))
    i = 15 - m0
    cge = Tstar + jnp.max(jnp.where(lane == m0, cs, 0))
    M = jnp.max(jnp.where(lane == i, v, 0))
    Bx = jstar * 16 + i
    return Bx, cge - M


def _sc_mask_body(keys_hbm, out_hbm, key_v, h1_v, cbuf_v):
    w = lax.axis_index("s") * 2 + lax.axis_index("c")

    @pl.when(w < 16)
    def _():
        zero16_i = jnp.zeros((16,), jnp.int32)
        ones16_i = jnp.ones((16,), jnp.int32)
        NV = HW // 16
        U = 8

        def zh(i, carry):
            h1_v[pl.ds(i * 16, 16)] = zero16_i
            return carry

        lax.fori_loop(0, NBKT // 16, zh, 0)

        pltpu.sync_copy(keys_hbm.at[w], key_v)

        def hpass1(jo, carry):
            for u in range(U):
                key = key_v[pl.ds((jo * U + u) * 16, 16)]
                bkt = (key >> jnp.uint32(BKT_SHIFT)).astype(jnp.int32)
                plsc.addupdate_scatter(h1_v, [bkt], ones16_i)
            return carry

        lax.fori_loop(0, NV // U, hpass1, 0)
        B1, gt1 = _scan_hist(h1_v, NBKT, K)

        def cpass(jo, off):
            for u in range(U):
                key = key_v[pl.ds((jo * U + u) * 16, 16)]
                m = (key >> jnp.uint32(BKT_SHIFT)).astype(jnp.int32) == B1
                cnt = plsc.all_reduce_population_count(m)[0]
                plsc.store_compressed(cbuf_v.at[pl.ds(off, 16)], key, mask=m)
                off = off + cnt
            return off

        M = lax.fori_loop(0, NV // U, cpass, jnp.int32(0))
        cbuf_v[pl.ds(M, 16)] = jnp.zeros((16,), jnp.uint32)

        nv = (M + 15) // 16
        prefix0 = B1.astype(jnp.uint32) << jnp.uint32(BKT_SHIFT)

        def refine(it, prefix):
            cand = prefix | (jnp.uint32(1) << (jnp.uint32(BKT_SHIFT - 1) -
                                               it.astype(jnp.uint32)))

            def cntloop(j, acc):
                kv = cbuf_v[pl.ds(j * 16, 16)]
                return acc + jnp.where(kv >= cand, 1, 0)

            accv = lax.fori_loop(0, nv, cntloop, jnp.zeros((16,), jnp.int32))
            cnt = jnp.sum(accv) + gt1
            return jnp.where(cnt >= K, cand, prefix)

        thr = lax.fori_loop(0, BKT_SHIFT, refine, prefix0)

        def mpass(jo, carry):
            for u in range(U):
                s = pl.ds((jo * U + u) * 16, 16)
                key = key_v[s]
                key_v[s] = jnp.where(key >= thr, jnp.uint32(ONE_F32_BITS),
                                     jnp.uint32(0))
            return carry

        lax.fori_loop(0, NV // U, mpass, 0)
        pltpu.sync_copy(key_v, out_hbm.at[w])


def _sc_masks(keys):
    f = pl.kernel(
        _sc_mask_body,
        out_type=jax.ShapeDtypeStruct((16, HW), jnp.uint32),
        mesh=plsc.VectorSubcoreMesh(core_axis_name="c", subcore_axis_name="s",
                                    num_cores=2, num_subcores=16),
        compiler_params=pltpu.CompilerParams(needs_layout_passes=False),
        scratch_types=[
            pltpu.VMEM((HW,), jnp.uint32),
            pltpu.VMEM((NBKT,), jnp.int32),
            pltpu.VMEM((HW + 16,), jnp.uint32),
        ],
    )
    return f(keys)


def _apply_body(mask_ref, x_ref, m_ref, o_ref):
    p = pl.program_id(0)
    b = p // 5
    l = p - b * 5
    xv = x_ref[...]
    mplane = m_ref[...]
    mv = (mask_ref[b, l] != 0).astype(jnp.float32)
    o_ref[...] = jnp.where(l == 0, xv, xv * (mplane * mv)[:, None, :, :])


def kernel(x, psm, mask):
    B, L, C, H, W = x.shape

    keys = pl.pallas_call(
        _key_body,
        grid=(B * (L - 1),),
        in_specs=[
            pl.BlockSpec((1, 1, 2, H, W), lambda p: (p // 4, p % 4 + 1, 0, 0, 0)),
            pl.BlockSpec((1, 1, 2, H, W), lambda p: (p // 4, 0, 0, 0, 0)),
        ],
        out_specs=pl.BlockSpec((1, H, W), lambda p: (p, 0, 0)),
        out_shape=jax.ShapeDtypeStruct((B * (L - 1), H, W), jnp.uint32),
    )(psm, psm)

    mask16 = _sc_masks(keys.reshape(B * (L - 1), H * W))
    mplane = lax.bitcast_convert_type(mask16, jnp.float32).reshape(16, H, W)

    CB = 64
    xf = x.reshape(B * L, C, H, W)
    out = pl.pallas_call(
        _apply_body,
        grid=(B * L, C // CB),
        in_specs=[
            pl.BlockSpec(memory_space=pltpu.SMEM),
            pl.BlockSpec((1, CB, H, W), lambda p, c: (p, c, 0, 0)),
            pl.BlockSpec((1, H, W),
                         lambda p, c: (jnp.clip(p - p // 5 - 1, 0, 15), 0, 0)),
        ],
        out_specs=pl.BlockSpec((1, CB, H, W), lambda p, c: (p, c, 0, 0)),
        out_shape=jax.ShapeDtypeStruct((B * L, C, H, W), jnp.float32),
    )(mask, xf, mplane)
    return out.reshape(B, L, C, H, W)


# speedup vs baseline: 1.1742x; 1.0592x over previous
import jax
import jax.numpy as jnp
from jax import lax
from jax.experimental import pallas as pl
from jax.experimental.pallas import tpu as pltpu
from jax.experimental.pallas import tpu_sc as plsc

K = 1024
HW = 128 * 256
NBKT = 4096
BKT_SHIFT = 20
ONE_F32_BITS = 0x3F800000


def _key_body(cav_ref, ego_ref, out_ref):
    cav = cav_ref[0, 0]
    ego = ego_ref[0, 0]
    r = jnp.maximum(cav[0] - ego[0], cav[1] - ego[1])
    bits = lax.bitcast_convert_type(r, jnp.uint32)
    neg = bits >= jnp.uint32(0x80000000)
    out_ref[0] = jnp.where(neg, ~bits, bits | jnp.uint32(0x80000000))


def _scan_hist(hist_ref, nbkt, need):
    lane = lax.iota(jnp.int32, 16)
    nvr = nbkt // 16
    ngrp = nvr // 16
    z = jnp.int32(0)

    def gstep(gg, carry):
        T, gstar, Tstar, found = carry
        g = (ngrp - 1) - gg
        acc = jnp.zeros((16,), jnp.int32)
        for t in range(16):
            acc = acc + hist_ref[pl.ds((g * 16 + t) * 16, 16)]
        s = jnp.sum(acc)
        cross = (1 - found) * jnp.where(T + s >= need, 1, 0)
        return (T + s,
                jnp.where(cross == 1, g, gstar),
                jnp.where(cross == 1, T, Tstar),
                jnp.where(T + s >= need, 1, found))

    _, gstar, Tg, _ = lax.fori_loop(0, ngrp, gstep, (z, z, z, z))

    def vstep(tt, carry):
        T, jstar, Tstar, found = carry
        j = gstar * 16 + (15 - tt)
        v = hist_ref[pl.ds(j * 16, 16)]
        s = jnp.sum(v)
        cross = (1 - found) * jnp.where(T + s >= need, 1, 0)
        return (T + s,
                jnp.where(cross == 1, j, jstar),
                jnp.where(cross == 1, T, Tstar),
                jnp.where(T + s >= need, 1, found))

    _, jstar, Tstar, _ = lax.fori_loop(0, 16, vstep, (Tg, z, z, z))

    v = hist_ref[pl.ds(jstar * 16, 16)]
    cs = plsc.cumsum(lax.rev(v, (0,)))
    m0 = jnp.min(jnp.where(cs >= need - Tstar, lane, 16))
    i = 15 - m0
    cge = Tstar + jnp.max(jnp.where(lane == m0, cs, 0))
    M = jnp.max(jnp.where(lane == i, v, 0))
    Bx = jstar * 16 + i
    return Bx, cge - M


HHW = HW // 2
CPAD = HHW + 128


def _sc_mask_body(keys_hbm, out_hbm, key_v, h1_v, h1b_v, cbuf_v, mvec_v,
                  sh_hist, sh_cbuf, sh_m):
    s = lax.axis_index("s")
    c = lax.axis_index("c")
    pair = c * 8 + s // 2
    half = s % 2

    zero16_i = jnp.zeros((16,), jnp.int32)
    ones16_i = jnp.ones((16,), jnp.int32)
    NV = HHW // 16
    U = 16

    def zh(i, carry):
        h1_v[pl.ds(i * 16, 16)] = zero16_i
        return carry

    lax.fori_loop(0, NBKT // 16, zh, 0)

    pltpu.sync_copy(keys_hbm.at[pair, pl.ds(half * HHW, HHW)], key_v)

    def hpass1(jo, carry):
        for u in range(U):
            key = key_v[pl.ds((jo * U + u) * 16, 16)]
            bkt = (key >> jnp.uint32(BKT_SHIFT)).astype(jnp.int32)
            plsc.addupdate_scatter(h1_v, [bkt], ones16_i)
        return carry

    lax.fori_loop(0, NV // U, hpass1, 0)

    pltpu.sync_copy(h1_v, sh_hist.at[s])
    plsc.subcore_barrier()
    pltpu.sync_copy(sh_hist.at[s - 2 * half + 1], h1b_v)

    def hmerge(i, carry):
        d = pl.ds(i * 16, 16)
        h1_v[d] = h1_v[d] + h1b_v[d]
        return carry

    lax.fori_loop(0, NBKT // 16, hmerge, 0)
    B1, gt1 = _scan_hist(h1_v, NBKT, K)

    def cpass(jo, off):
        for u in range(U):
            key = key_v[pl.ds((jo * U + u) * 16, 16)]
            m = (key >> jnp.uint32(BKT_SHIFT)).astype(jnp.int32) == B1
            cnt = plsc.all_reduce_population_count(m)[0]
            plsc.store_compressed(cbuf_v.at[pl.ds(off, 16)], key, mask=m)
            off = off + cnt
        return off

    M = lax.fori_loop(0, NV // U, cpass, jnp.int32(0))
    cbuf_v[pl.ds(M, 16)] = jnp.zeros((16,), jnp.uint32)

    pltpu.sync_copy(cbuf_v.at[pl.ds(0, CPAD)], sh_cbuf.at[s])
    mvec_v[pl.ds(0, 16)] = M + zero16_i
    pltpu.sync_copy(mvec_v, sh_m.at[s])
    plsc.subcore_barrier()
    pltpu.sync_copy(sh_m.at[s - 2 * half + 1], mvec_v)
    plsc.subcore_barrier()
    pltpu.sync_copy(sh_cbuf.at[s - 2 * half + 1],
                    cbuf_v.at[pl.ds(CPAD, CPAD)])
    Mb = mvec_v[pl.ds(0, 16)][0]

    nva = (M + 15) // 16
    nvb = (Mb + 15) // 16
    prefix0 = B1.astype(jnp.uint32) << jnp.uint32(BKT_SHIFT)

    def refine(it, prefix):
        cand = prefix | (jnp.uint32(1) << (jnp.uint32(BKT_SHIFT - 1) -
                                           it.astype(jnp.uint32)))

        def cntloop(j, acc):
            kv = cbuf_v[pl.ds(j * 16, 16)]
            return acc + jnp.where(kv >= cand, 1, 0)

        def cntloop_b(j, acc):
            kv = cbuf_v[pl.ds(CPAD + j * 16, 16)]
            return acc + jnp.where(kv >= cand, 1, 0)

        accv = lax.fori_loop(0, nva, cntloop, jnp.zeros((16,), jnp.int32))
        accv = lax.fori_loop(0, nvb, cntloop_b, accv)
        cnt = jnp.sum(accv) + gt1
        return jnp.where(cnt >= K, cand, prefix)

    thr = lax.fori_loop(0, BKT_SHIFT, refine, prefix0)

    def mpass(jo, carry):
        for u in range(U):
            d = pl.ds((jo * U + u) * 16, 16)
            key = key_v[d]
            key_v[d] = jnp.where(key >= thr, jnp.uint32(ONE_F32_BITS),
                                 jnp.uint32(0))
        return carry

    lax.fori_loop(0, NV // U, mpass, 0)
    pltpu.sync_copy(key_v, out_hbm.at[pair, pl.ds(half * HHW, HHW)])


def _sc_masks(keys):
    f = pl.kernel(
        _sc_mask_body,
        out_type=jax.ShapeDtypeStruct((16, HW), jnp.uint32),
        mesh=plsc.VectorSubcoreMesh(core_axis_name="c", subcore_axis_name="s",
                                    num_cores=2, num_subcores=16),
        compiler_params=pltpu.CompilerParams(needs_layout_passes=False),
        scratch_types=[
            pltpu.VMEM((HHW,), jnp.uint32),
            pltpu.VMEM((NBKT,), jnp.int32),
            pltpu.VMEM((NBKT,), jnp.int32),
            pltpu.VMEM((2 * CPAD,), jnp.uint32),
            pltpu.VMEM((128,), jnp.int32),
            pltpu.VMEM_SHARED((16, NBKT), jnp.int32),
            pltpu.VMEM_SHARED((16, CPAD), jnp.uint32),
            pltpu.VMEM_SHARED((16, 128), jnp.int32),
        ],
    )
    return f(keys)


def _apply_body(mask_ref, x_ref, m_ref, o_ref):
    p = pl.program_id(0)
    b = p // 5
    l = p - b * 5
    xv = x_ref[...]
    mplane = m_ref[...]
    mv = (mask_ref[b, l] != 0).astype(jnp.float32)
    o_ref[...] = jnp.where(l == 0, xv, xv * (mplane * mv)[:, None, :, :])


def kernel(x, psm, mask):
    B, L, C, H, W = x.shape

    keys = pl.pallas_call(
        _key_body,
        grid=(B * (L - 1),),
        in_specs=[
            pl.BlockSpec((1, 1, 2, H, W), lambda p: (p // 4, p % 4 + 1, 0, 0, 0)),
            pl.BlockSpec((1, 1, 2, H, W), lambda p: (p // 4, 0, 0, 0, 0)),
        ],
        out_specs=pl.BlockSpec((1, H, W), lambda p: (p, 0, 0)),
        out_shape=jax.ShapeDtypeStruct((B * (L - 1), H, W), jnp.uint32),
    )(psm, psm)

    mask16 = _sc_masks(keys.reshape(B * (L - 1), H * W))
    mplane = lax.bitcast_convert_type(mask16, jnp.float32).reshape(16, H, W)

    CB = 64
    xf = x.reshape(B * L, C, H, W)
    out = pl.pallas_call(
        _apply_body,
        grid=(B * L, C // CB),
        in_specs=[
            pl.BlockSpec(memory_space=pltpu.SMEM),
            pl.BlockSpec((1, CB, H, W), lambda p, c: (p, c, 0, 0)),
            pl.BlockSpec((1, H, W),
                         lambda p, c: (jnp.clip(p - p // 5 - 1, 0, 15), 0, 0)),
        ],
        out_specs=pl.BlockSpec((1, CB, H, W), lambda p, c: (p, c, 0, 0)),
        out_shape=jax.ShapeDtypeStruct((B * L, C, H, W), jnp.float32),
    )(mask, xf, mplane)
    return out.reshape(B, L, C, H, W)
